# baseline (speedup 1.0000x reference)
import functools

import jax
import jax.numpy as jnp
from jax import lax
from jax.experimental import pallas as pl
from jax.experimental.pallas import tpu as pltpu
from jax.experimental.pallas import tpu_sc as plsc

P = 16
D = 64
DP = 128
K = 1024
TOK_BLK = 1152

_NC = 2
_NS = 16
_NW = _NC * _NS
_TOK_TOTAL = 9216
_BPW = _TOK_TOTAL // _NW
_CHUNK = 96
_NCHUNK = _BPW // _CHUNK


def _encode_argmin_body(p_ref, w_ref, b_ref, cb_ref, cn_ref, z_ref, idx_ref):
    z = jnp.dot(p_ref[...], w_ref[...],
                preferred_element_type=jnp.float32) + b_ref[...]
    z_ref[...] = z
    d = cn_ref[...] - lax.dot_general(
        z, cb_ref[...], (((1,), (1,)), ((), ())),
        preferred_element_type=jnp.float32)
    dmin = jnp.min(d, axis=1, keepdims=True)
    iota = lax.broadcasted_iota(jnp.int32, d.shape, 1)
    idx = jnp.min(jnp.where(d == dmin, iota, K), axis=1)
    idx_ref[...] = idx[None, None, :]


def _decode_loss_body(p_ref, z_ref, emb_ref, wd_ref, bd_ref,
                      recon_ref, vq_ref, mse_ref):
    emb = emb_ref[:, :D]
    recon = jnp.dot(emb, wd_ref[...],
                    preferred_element_type=jnp.float32) + bd_ref[...]
    recon_ref[...] = recon
    dz = z_ref[...] - emb
    vq_p = jnp.sum(jnp.sum(dz * dz, axis=1, keepdims=True), axis=0,
                   keepdims=True)
    dr = recon - p_ref[...]
    mse_p = jnp.sum(jnp.sum(dr * dr, axis=1, keepdims=True), axis=0,
                    keepdims=True)

    @pl.when(pl.program_id(0) == 0)
    def _init():
        vq_ref[...] = vq_p
        mse_ref[...] = mse_p

    @pl.when(pl.program_id(0) != 0)
    def _acc():
        vq_ref[...] += vq_p
        mse_ref[...] += mse_p


@functools.partial(
    pl.kernel,
    mesh=plsc.VectorSubcoreMesh(core_axis_name="c", subcore_axis_name="s"),
    out_type=jax.ShapeDtypeStruct((_TOK_TOTAL, DP), jnp.float32),
    scratch_types=[
        pltpu.VMEM((_NCHUNK, _CHUNK), jnp.int32),
        pltpu.VMEM((_BPW, DP), jnp.float32),
        pltpu.SemaphoreType.DMA,
    ],
)
def _sc_gather(cb_hbm, idx_hbm, out_hbm, idx_v, rows_v, sem):
    wid = lax.axis_index("s") * _NC + lax.axis_index("c")
    base = wid * _BPW
    pltpu.sync_copy(idx_hbm.at[wid], idx_v)
    for j in range(_NCHUNK):
        pltpu.async_copy(cb_hbm.at[idx_v.at[j]],
                         rows_v.at[pl.ds(j * _CHUNK, _CHUNK)], sem).wait()
    pltpu.sync_copy(rows_v, out_hbm.at[pl.ds(base, _BPW)])


def kernel(inputs, W_enc, b_enc, codebook, W_dec, b_dec):
    Bb, Cc, H, W = inputs.shape
    h, w = H // P, W // P
    T = Bb * h * w
    pd = Cc * P * P
    cn_half = 0.5 * jnp.sum(codebook * codebook, axis=1)[None, :]
    cb_p = jnp.pad(codebook, ((0, 0), (0, DP - D)))
    patches = inputs.reshape(Bb, Cc, h, P, w, P).transpose(
        0, 2, 4, 1, 3, 5).reshape(T, pd)
    G = T // TOK_BLK

    z, idx3 = pl.pallas_call(
        _encode_argmin_body,
        grid=(G,),
        in_specs=[
            pl.BlockSpec((TOK_BLK, pd), lambda i: (i, 0)),
            pl.BlockSpec((pd, D), lambda i: (0, 0)),
            pl.BlockSpec((1, D), lambda i: (0, 0)),
            pl.BlockSpec((K, D), lambda i: (0, 0)),
            pl.BlockSpec((1, K), lambda i: (0, 0)),
        ],
        out_specs=[
            pl.BlockSpec((TOK_BLK, D), lambda i: (i, 0)),
            pl.BlockSpec((1, 1, TOK_BLK), lambda i: (i, 0, 0)),
        ],
        out_shape=[
            jax.ShapeDtypeStruct((T, D), jnp.float32),
            jax.ShapeDtypeStruct((G, 1, TOK_BLK), jnp.int32),
        ],
    )(patches, W_enc, b_enc.reshape(1, D), codebook, cn_half)

    idx = idx3.reshape(_NW, _NCHUNK, _CHUNK)
    emb = _sc_gather(cb_p, idx)

    recon_p, vq_sse, mse_sse = pl.pallas_call(
        _decode_loss_body,
        grid=(G,),
        in_specs=[
            pl.BlockSpec((TOK_BLK, pd), lambda i: (i, 0)),
            pl.BlockSpec((TOK_BLK, D), lambda i: (i, 0)),
            pl.BlockSpec((TOK_BLK, DP), lambda i: (i, 0)),
            pl.BlockSpec((D, pd), lambda i: (0, 0)),
            pl.BlockSpec((1, pd), lambda i: (0, 0)),
        ],
        out_specs=[
            pl.BlockSpec((TOK_BLK, pd), lambda i: (i, 0)),
            pl.BlockSpec((1, 1), lambda i: (0, 0)),
            pl.BlockSpec((1, 1), lambda i: (0, 0)),
        ],
        out_shape=[
            jax.ShapeDtypeStruct((T, pd), jnp.float32),
            jax.ShapeDtypeStruct((1, 1), jnp.float32),
            jax.ShapeDtypeStruct((1, 1), jnp.float32),
        ],
    )(patches, z, emb, W_dec, b_dec.reshape(1, pd))

    recon = recon_p.reshape(Bb, h, w, Cc, P, P).transpose(
        0, 3, 1, 4, 2, 5).reshape(Bb, Cc, H, W)
    mse = mse_sse[0, 0] / (Bb * Cc * H * W)
    vq = vq_sse[0, 0] / (T * D)
    loss = 1.25 * vq + mse
    return (loss, mse, recon)


# speedup vs baseline: 2.5381x; 2.5381x over previous
import functools

import jax
import jax.numpy as jnp
from jax import lax
from jax.experimental import pallas as pl
from jax.experimental.pallas import tpu as pltpu
from jax.experimental.pallas import tpu_sc as plsc

P = 16
D = 64
DP = 128
K = 1024

_NC = 2
_NS = 16
_NW = _NC * _NS
_TOK_TOTAL = 9216
_BPW = _TOK_TOTAL // _NW
_CHUNK = 96
_NCHUNK = _BPW // _CHUNK


def _patchify_block(x):
    t = x.reshape(3, 24, P, 24, P).transpose(1, 3, 0, 2, 4)
    return t.reshape(576, 3 * P * P)


def _unpatchify_block(t):
    x = t.reshape(24, 24, 3, P, P).transpose(2, 0, 3, 1, 4)
    return x.reshape(3, 384, 384)


def _encode_argmin_body(x_ref, w_ref, b_ref, cb_ref, cn_ref, z_ref, idx_ref):
    patches = _patchify_block(x_ref[0])
    z = jnp.dot(patches, w_ref[...],
                preferred_element_type=jnp.float32) + b_ref[...]
    z_ref[...] = z
    d = cn_ref[...] - lax.dot_general(
        z, cb_ref[...], (((1,), (1,)), ((), ())),
        preferred_element_type=jnp.float32)
    dmin = jnp.min(d, axis=1, keepdims=True)
    iota = lax.broadcasted_iota(jnp.int32, d.shape, 1)
    idx = jnp.min(jnp.where(d == dmin, iota, K), axis=1)
    idx_ref[...] = idx[None, None, :]


def _decode_loss_body(x_ref, z_ref, emb_ref, wd_ref, bd_ref,
                      recon_ref, vq_ref, mse_ref):
    emb = emb_ref[:, :D]
    recon = jnp.dot(emb, wd_ref[...],
                    preferred_element_type=jnp.float32) + bd_ref[...]
    rimg = _unpatchify_block(recon)
    recon_ref[0] = rimg
    dz = z_ref[...] - emb
    vq_ref[...] = jnp.sum(dz * dz).reshape(1, 1, 1)
    dr = rimg - x_ref[0]
    mse_ref[...] = jnp.sum(dr * dr).reshape(1, 1, 1)


@functools.partial(
    pl.kernel,
    mesh=plsc.VectorSubcoreMesh(core_axis_name="c", subcore_axis_name="s"),
    out_type=jax.ShapeDtypeStruct((_TOK_TOTAL, DP), jnp.float32),
    scratch_types=[
        pltpu.VMEM((_NCHUNK, _CHUNK), jnp.int32),
        pltpu.VMEM((_BPW, DP), jnp.float32),
        pltpu.SemaphoreType.DMA,
    ],
)
def _sc_gather(cb_hbm, idx_hbm, out_hbm, idx_v, rows_v, sem):
    wid = lax.axis_index("s") * _NC + lax.axis_index("c")
    base = wid * _BPW
    pltpu.sync_copy(idx_hbm.at[wid], idx_v)
    for j in range(_NCHUNK):
        pltpu.async_copy(cb_hbm.at[idx_v.at[j]],
                         rows_v.at[pl.ds(j * _CHUNK, _CHUNK)], sem).wait()
    pltpu.sync_copy(rows_v, out_hbm.at[pl.ds(base, _BPW)])


def kernel(inputs, W_enc, b_enc, codebook, W_dec, b_dec):
    Bb, Cc, H, W = inputs.shape
    h, w = H // P, W // P
    T = Bb * h * w
    pd = Cc * P * P
    ntok = h * w
    cn_half = 0.5 * jnp.sum(codebook * codebook, axis=1)[None, :]
    cb_p = jnp.pad(codebook, ((0, 0), (0, DP - D)))

    z, idx3 = pl.pallas_call(
        _encode_argmin_body,
        grid=(Bb,),
        in_specs=[
            pl.BlockSpec((1, Cc, H, W), lambda i: (i, 0, 0, 0)),
            pl.BlockSpec((pd, D), lambda i: (0, 0)),
            pl.BlockSpec((1, D), lambda i: (0, 0)),
            pl.BlockSpec((K, D), lambda i: (0, 0)),
            pl.BlockSpec((1, K), lambda i: (0, 0)),
        ],
        out_specs=[
            pl.BlockSpec((ntok, D), lambda i: (i, 0)),
            pl.BlockSpec((1, 1, ntok), lambda i: (i, 0, 0)),
        ],
        out_shape=[
            jax.ShapeDtypeStruct((T, D), jnp.float32),
            jax.ShapeDtypeStruct((Bb, 1, ntok), jnp.int32),
        ],
        compiler_params=pltpu.CompilerParams(
            dimension_semantics=("parallel",)),
    )(inputs, W_enc, b_enc.reshape(1, D), codebook, cn_half)

    idx = idx3.reshape(_NW, _NCHUNK, _CHUNK)
    emb = _sc_gather(cb_p, idx)

    recon, vq_parts, mse_parts = pl.pallas_call(
        _decode_loss_body,
        grid=(Bb,),
        in_specs=[
            pl.BlockSpec((1, Cc, H, W), lambda i: (i, 0, 0, 0)),
            pl.BlockSpec((ntok, D), lambda i: (i, 0)),
            pl.BlockSpec((ntok, DP), lambda i: (i, 0)),
            pl.BlockSpec((D, pd), lambda i: (0, 0)),
            pl.BlockSpec((1, pd), lambda i: (0, 0)),
        ],
        out_specs=[
            pl.BlockSpec((1, Cc, H, W), lambda i: (i, 0, 0, 0)),
            pl.BlockSpec((1, 1, 1), lambda i: (i, 0, 0)),
            pl.BlockSpec((1, 1, 1), lambda i: (i, 0, 0)),
        ],
        out_shape=[
            jax.ShapeDtypeStruct((Bb, Cc, H, W), jnp.float32),
            jax.ShapeDtypeStruct((Bb, 1, 1), jnp.float32),
            jax.ShapeDtypeStruct((Bb, 1, 1), jnp.float32),
        ],
        compiler_params=pltpu.CompilerParams(
            dimension_semantics=("parallel",)),
    )(inputs, z, emb, W_dec, b_dec.reshape(1, pd))

    mse = jnp.sum(mse_parts) / (Bb * Cc * H * W)
    vq = jnp.sum(vq_parts) / (T * D)
    loss = 1.25 * vq + mse
    return (loss, mse, recon)


# speedup vs baseline: 2.6256x; 1.0345x over previous
import functools

import jax
import jax.numpy as jnp
from jax import lax
from jax.experimental import pallas as pl
from jax.experimental.pallas import tpu as pltpu
from jax.experimental.pallas import tpu_sc as plsc

P = 16
D = 64
DP = 128
K = 1024

_NC = 2
_NS = 16
_NW = _NC * _NS
_TOK_TOTAL = 9216
_BPW = _TOK_TOTAL // _NW
_CHUNK = 96
_NCHUNK = _BPW // _CHUNK


def _patchify_block(x):
    t = x.reshape(3, 24, P, 24, P).transpose(1, 3, 0, 2, 4)
    return t.reshape(576, 3 * P * P)


def _unpatchify_block(t):
    x = t.reshape(24, 24, 3, P, P).transpose(2, 0, 3, 1, 4)
    return x.reshape(3, 384, 384)


def _encode_argmin_body(x_ref, w_ref, b_ref, cb_ref, cn_ref, z_ref, idx_ref):
    z = b_ref[...]
    for c in range(3):
        pc_ = x_ref[0, c].reshape(24, P, 24, P).transpose(0, 2, 1, 3)
        z = z + jnp.dot(pc_.reshape(576, P * P),
                        w_ref[pl.ds(c * P * P, P * P), :],
                        preferred_element_type=jnp.float32)
    z_ref[...] = z
    d = cn_ref[...] - lax.dot_general(
        z, cb_ref[...], (((1,), (1,)), ((), ())),
        preferred_element_type=jnp.float32)
    dmin = jnp.min(d, axis=1, keepdims=True)
    iota = lax.broadcasted_iota(jnp.int32, d.shape, 1)
    idx = jnp.min(jnp.where(d == dmin, iota, K), axis=1)
    idx_ref[...] = idx[None, None, :]


def _decode_loss_body(x_ref, z_ref, emb_ref, wd_ref, bd_ref,
                      recon_ref, vq_ref, mse_ref):
    emb = emb_ref[:, :D]
    recon = jnp.dot(emb, wd_ref[...],
                    preferred_element_type=jnp.float32) + bd_ref[...]
    msse = jnp.zeros((), jnp.float32)
    for c in range(3):
        rc = recon[:, c * P * P:(c + 1) * P * P].reshape(
            24, 24, P, P).transpose(0, 2, 1, 3).reshape(384, 384)
        recon_ref[0, c] = rc
        dr = rc - x_ref[0, c]
        msse = msse + jnp.sum(dr * dr)
    dz = z_ref[...] - emb
    vq_ref[...] = jnp.sum(dz * dz).reshape(1, 1, 1)
    mse_ref[...] = msse.reshape(1, 1, 1)


@functools.partial(
    pl.kernel,
    mesh=plsc.VectorSubcoreMesh(core_axis_name="c", subcore_axis_name="s"),
    out_type=jax.ShapeDtypeStruct((_TOK_TOTAL, DP), jnp.float32),
    scratch_types=[
        pltpu.VMEM((_NCHUNK, _CHUNK), jnp.int32),
        pltpu.VMEM((_BPW, DP), jnp.float32),
        pltpu.SemaphoreType.DMA,
    ],
)
def _sc_gather(cb_hbm, idx_hbm, out_hbm, idx_v, rows_v, sem):
    wid = lax.axis_index("s") * _NC + lax.axis_index("c")
    base = wid * _BPW
    pltpu.sync_copy(idx_hbm.at[wid], idx_v)
    for j in range(_NCHUNK):
        pltpu.async_copy(cb_hbm.at[idx_v.at[j]],
                         rows_v.at[pl.ds(j * _CHUNK, _CHUNK)], sem).wait()
    pltpu.sync_copy(rows_v, out_hbm.at[pl.ds(base, _BPW)])


def kernel(inputs, W_enc, b_enc, codebook, W_dec, b_dec):
    Bb, Cc, H, W = inputs.shape
    h, w = H // P, W // P
    T = Bb * h * w
    pd = Cc * P * P
    ntok = h * w
    cn_half = 0.5 * jnp.sum(codebook * codebook, axis=1)[None, :]
    cb_p = jnp.pad(codebook, ((0, 0), (0, DP - D)))

    z, idx3 = pl.pallas_call(
        _encode_argmin_body,
        grid=(Bb,),
        in_specs=[
            pl.BlockSpec((1, Cc, H, W), lambda i: (i, 0, 0, 0)),
            pl.BlockSpec((pd, D), lambda i: (0, 0)),
            pl.BlockSpec((1, D), lambda i: (0, 0)),
            pl.BlockSpec((K, D), lambda i: (0, 0)),
            pl.BlockSpec((1, K), lambda i: (0, 0)),
        ],
        out_specs=[
            pl.BlockSpec((ntok, D), lambda i: (i, 0)),
            pl.BlockSpec((1, 1, ntok), lambda i: (i, 0, 0)),
        ],
        out_shape=[
            jax.ShapeDtypeStruct((T, D), jnp.float32),
            jax.ShapeDtypeStruct((Bb, 1, ntok), jnp.int32),
        ],
        compiler_params=pltpu.CompilerParams(
            dimension_semantics=("parallel",)),
    )(inputs, W_enc, b_enc.reshape(1, D), codebook, cn_half)

    idx = idx3.reshape(_NW, _NCHUNK, _CHUNK)
    emb = _sc_gather(cb_p, idx)

    recon, vq_parts, mse_parts = pl.pallas_call(
        _decode_loss_body,
        grid=(Bb,),
        in_specs=[
            pl.BlockSpec((1, Cc, H, W), lambda i: (i, 0, 0, 0)),
            pl.BlockSpec((ntok, D), lambda i: (i, 0)),
            pl.BlockSpec((ntok, DP), lambda i: (i, 0)),
            pl.BlockSpec((D, pd), lambda i: (0, 0)),
            pl.BlockSpec((1, pd), lambda i: (0, 0)),
        ],
        out_specs=[
            pl.BlockSpec((1, Cc, H, W), lambda i: (i, 0, 0, 0)),
            pl.BlockSpec((1, 1, 1), lambda i: (i, 0, 0)),
            pl.BlockSpec((1, 1, 1), lambda i: (i, 0, 0)),
        ],
        out_shape=[
            jax.ShapeDtypeStruct((Bb, Cc, H, W), jnp.float32),
            jax.ShapeDtypeStruct((Bb, 1, 1), jnp.float32),
            jax.ShapeDtypeStruct((Bb, 1, 1), jnp.float32),
        ],
        compiler_params=pltpu.CompilerParams(
            dimension_semantics=("parallel",)),
    )(inputs, z, emb, W_dec, b_dec.reshape(1, pd))

    mse = jnp.sum(mse_parts) / (Bb * Cc * H * W)
    vq = jnp.sum(vq_parts) / (T * D)
    loss = 1.25 * vq + mse
    return (loss, mse, recon)
